# baseline (speedup 1.0000x reference)
import functools

import jax
import jax.numpy as jnp
from jax import lax
from jax.experimental import pallas as pl
from jax.experimental.pallas import tpu as pltpu
from jax.experimental.pallas import tpu_sc as plsc

_K = 3
_M = 4
_L = 16


def _build_adj_t(ei, n, e):
    info = plsc.get_sparse_core_info()
    nw = info.num_cores * info.num_subcores
    rows = n // nw
    mesh = plsc.VectorSubcoreMesh(core_axis_name="c", subcore_axis_name="s")

    @functools.partial(
        pl.kernel,
        mesh=mesh,
        out_type=jax.ShapeDtypeStruct((n, n), jnp.float32),
        scratch_types=[
            pltpu.VMEM((e,), jnp.int32),
            pltpu.VMEM((e,), jnp.int32),
            pltpu.VMEM((rows, n), jnp.float32),
        ],
        compiler_params=pltpu.CompilerParams(needs_layout_passes=False),
    )
    def sc_scatter(ei_hbm, t_hbm, src_v, dst_v, slab):
        wid = lax.axis_index("s") * info.num_cores + lax.axis_index("c")
        base = wid * rows
        pltpu.sync_copy(ei_hbm.at[0], src_v)
        pltpu.sync_copy(ei_hbm.at[1], dst_v)

        zeros = jnp.zeros((_L,), jnp.float32)
        npl = n // _L

        @plsc.parallel_loop(0, rows * npl, 1, unroll=2)
        def _(j):
            slab[j // npl, pl.ds((j % npl) * _L, _L)] = zeros

        ones = jnp.ones((_L,), jnp.float32)
        lanes = lax.iota(jnp.int32, _L)

        for r0 in range(0, rows, _L):
            plsc.store_scatter(slab, [r0 + lanes, base + r0 + lanes], ones)

        @plsc.parallel_loop(0, e // _L, 1, unroll=2)
        def _(i):
            s = src_v[pl.ds(i * _L, _L)]
            d = dst_v[pl.ds(i * _L, _L)]
            m = (d >= base) & (d < base + rows)
            r = jnp.where(m, d - base, 0)
            plsc.store_scatter(slab, [r, s], ones, mask=m)

        pltpu.sync_copy(slab, t_hbm.at[pl.ds(base, rows)])

    return sc_scatter(ei)


def _tc_body(t_ref, x_ref, out_ref):
    d = x_ref.shape[1]

    xa = jnp.abs(x_ref[...])
    x2 = xa * xa
    xcat = jnp.concatenate([xa, x2, x2 * xa, x2 * x2], axis=1)
    xcat = xcat.astype(jnp.bfloat16)

    t1 = t_ref[...].astype(jnp.bfloat16)
    c2 = lax.dot_general(t1, t1, (((1,), (0,)), ((), ())),
                         preferred_element_type=jnp.float32)
    t2 = (c2 > 0.0).astype(jnp.bfloat16)
    c3 = lax.dot_general(t1, t2, (((1,), (0,)), ((), ())),
                         preferred_element_type=jnp.float32)
    t3 = (c3 > 0.0).astype(jnp.bfloat16)

    for k, t in enumerate((t1, t2, t3)):
        ok = lax.dot_general(t, xcat, (((1,), (0,)), ((), ())),
                             preferred_element_type=jnp.float32)
        for m in range(_M):
            out_ref[0, :, k, m, :] = ok[:, m * d:(m + 1) * d]


def _tc_compute(t, x2d, n, d):
    return pl.pallas_call(
        _tc_body,
        out_shape=jax.ShapeDtypeStruct((1, n, _K, _M, d), jnp.float32),
    )(t, x2d)


def kernel(x, edge_index):
    b, n, d = x.shape
    e = edge_index.shape[1]
    t = _build_adj_t(edge_index, n, e)
    outs = [_tc_compute(t, x[bi], n, d) for bi in range(b)]
    if b == 1:
        return outs[0]
    return jnp.concatenate(outs, axis=0)


# speedup vs baseline: 1.0635x; 1.0635x over previous
import functools

import jax
import jax.numpy as jnp
from jax import lax
from jax.experimental import pallas as pl
from jax.experimental.pallas import tpu as pltpu
from jax.experimental.pallas import tpu_sc as plsc

_K = 3
_M = 4
_L = 16


def _build_adj_t(ei, n, e):
    info = plsc.get_sparse_core_info()
    nw = info.num_cores * info.num_subcores
    rows = n // nw
    mesh = plsc.VectorSubcoreMesh(core_axis_name="c", subcore_axis_name="s")

    @functools.partial(
        pl.kernel,
        mesh=mesh,
        out_type=jax.ShapeDtypeStruct((n, n), jnp.float32),
        scratch_types=[
            pltpu.VMEM((e,), jnp.int32),
            pltpu.VMEM((e,), jnp.int32),
            pltpu.VMEM((rows, n), jnp.float32),
        ],
        compiler_params=pltpu.CompilerParams(needs_layout_passes=False),
    )
    def sc_scatter(src_hbm, dst_hbm, t_hbm, src_v, dst_v, slab):
        wid = lax.axis_index("s") * info.num_cores + lax.axis_index("c")
        base = wid * rows
        pltpu.sync_copy(src_hbm, src_v)
        pltpu.sync_copy(dst_hbm, dst_v)

        zeros = jnp.zeros((_L,), jnp.float32)
        npl = n // _L

        @plsc.parallel_loop(0, rows * npl, 1, unroll=4)
        def _(j):
            slab[j // npl, pl.ds((j % npl) * _L, _L)] = zeros

        ones = jnp.ones((_L,), jnp.float32)
        lanes = lax.iota(jnp.int32, _L)

        for r0 in range(0, rows, _L):
            plsc.store_scatter(slab, [r0 + lanes, base + r0 + lanes], ones)

        @plsc.parallel_loop(0, e // _L, 1, unroll=4)
        def _(i):
            s = src_v[pl.ds(i * _L, _L)]
            d = dst_v[pl.ds(i * _L, _L)]
            m = (d >= base) & (d < base + rows)
            r = jnp.where(m, d - base, 0)
            plsc.store_scatter(slab, [r, s], ones, mask=m)

        pltpu.sync_copy(slab, t_hbm.at[pl.ds(base, rows)])

    return sc_scatter(ei[0], ei[1])


def _tc_body(t_ref, x_ref, out_ref):
    d = x_ref.shape[1]

    xa = jnp.abs(x_ref[...])
    x2 = xa * xa
    xcat = jnp.concatenate([xa, x2, x2 * xa, x2 * x2], axis=1)
    xcat = xcat.astype(jnp.bfloat16)

    t1 = t_ref[...].astype(jnp.bfloat16)
    c2 = lax.dot_general(t1, t1, (((1,), (0,)), ((), ())),
                         preferred_element_type=jnp.float32)
    t2 = (c2 > 0.0).astype(jnp.bfloat16)
    c3 = lax.dot_general(t1, t2, (((1,), (0,)), ((), ())),
                         preferred_element_type=jnp.float32)
    t3 = (c3 > 0.0).astype(jnp.bfloat16)

    for k, t in enumerate((t1, t2, t3)):
        ok = lax.dot_general(t, xcat, (((1,), (0,)), ((), ())),
                             preferred_element_type=jnp.float32)
        for m in range(_M):
            out_ref[0, :, k, m, :] = ok[:, m * d:(m + 1) * d]


def _tc_compute(t, x2d, n, d):
    return pl.pallas_call(
        _tc_body,
        out_shape=jax.ShapeDtypeStruct((1, n, _K, _M, d), jnp.float32),
    )(t, x2d)


def kernel(x, edge_index):
    b, n, d = x.shape
    e = edge_index.shape[1]
    t = _build_adj_t(edge_index, n, e)
    outs = [_tc_compute(t, x[bi], n, d) for bi in range(b)]
    if b == 1:
        return outs[0]
    return jnp.concatenate(outs, axis=0)


# speedup vs baseline: 1.1295x; 1.0621x over previous
import functools

import jax
import jax.numpy as jnp
from jax import lax
from jax.experimental import pallas as pl
from jax.experimental.pallas import tpu as pltpu
from jax.experimental.pallas import tpu_sc as plsc

_K = 3
_M = 4
_L = 16


def _build_adj_t(ei, n, e):
    info = plsc.get_sparse_core_info()
    nw = info.num_cores * info.num_subcores
    rows = n // nw
    mesh = plsc.VectorSubcoreMesh(core_axis_name="c", subcore_axis_name="s")

    @functools.partial(
        pl.kernel,
        mesh=mesh,
        out_type=jax.ShapeDtypeStruct((n, n), jnp.float32),
        scratch_types=[
            pltpu.VMEM((e,), jnp.int32),
            pltpu.VMEM((e,), jnp.int32),
            pltpu.VMEM((rows, n), jnp.float32),
            pltpu.SemaphoreType.DMA,
            pltpu.SemaphoreType.DMA,
        ],
        compiler_params=pltpu.CompilerParams(needs_layout_passes=False),
    )
    def sc_scatter(src_hbm, dst_hbm, t_hbm, src_v, dst_v, slab, sem1, sem2):
        wid = lax.axis_index("s") * info.num_cores + lax.axis_index("c")
        base = wid * rows
        cp1 = pltpu.async_copy(src_hbm, src_v, sem1)
        cp2 = pltpu.async_copy(dst_hbm, dst_v, sem2)

        zeros = jnp.zeros((_L,), jnp.float32)
        npl = n // _L

        @plsc.parallel_loop(0, rows * npl, 1, unroll=4)
        def _(j):
            slab[j // npl, pl.ds((j % npl) * _L, _L)] = zeros

        cp1.wait()
        cp2.wait()

        ones = jnp.ones((_L,), jnp.float32)
        lanes = lax.iota(jnp.int32, _L)

        for r0 in range(0, rows, _L):
            plsc.store_scatter(slab, [r0 + lanes, base + r0 + lanes], ones)

        @plsc.parallel_loop(0, e // _L, 1, unroll=4)
        def _(i):
            s = src_v[pl.ds(i * _L, _L)]
            d = dst_v[pl.ds(i * _L, _L)]
            m = (d >= base) & (d < base + rows)
            r = jnp.where(m, d - base, 0)
            plsc.store_scatter(slab, [r, s], ones, mask=m)

        pltpu.sync_copy(slab, t_hbm.at[pl.ds(base, rows)])

    return sc_scatter(ei[0], ei[1])


def _tc_body(t_ref, x_ref, out_ref):
    d = x_ref.shape[1]

    xa = jnp.abs(x_ref[...])
    x2 = xa * xa
    xcat = jnp.concatenate([xa, x2, x2 * xa, x2 * x2], axis=1)
    xcat = xcat.astype(jnp.bfloat16)

    t1 = t_ref[...].astype(jnp.bfloat16)
    c2 = lax.dot_general(t1, t1, (((1,), (0,)), ((), ())),
                         preferred_element_type=jnp.float32)
    t2 = (c2 > 0.0).astype(jnp.bfloat16)
    c3 = lax.dot_general(t1, t2, (((1,), (0,)), ((), ())),
                         preferred_element_type=jnp.float32)
    t3 = (c3 > 0.0).astype(jnp.bfloat16)

    for k, t in enumerate((t1, t2, t3)):
        ok = lax.dot_general(t, xcat, (((1,), (0,)), ((), ())),
                             preferred_element_type=jnp.float32)
        for m in range(_M):
            out_ref[0, :, k, m, :] = ok[:, m * d:(m + 1) * d]


def _tc_compute(t, x2d, n, d):
    return pl.pallas_call(
        _tc_body,
        out_shape=jax.ShapeDtypeStruct((1, n, _K, _M, d), jnp.float32),
    )(t, x2d)


def kernel(x, edge_index):
    b, n, d = x.shape
    e = edge_index.shape[1]
    t = _build_adj_t(edge_index, n, e)
    outs = [_tc_compute(t, x[bi], n, d) for bi in range(b)]
    if b == 1:
        return outs[0]
    return jnp.concatenate(outs, axis=0)


# speedup vs baseline: 1.1446x; 1.0134x over previous
import functools

import jax
import jax.numpy as jnp
from jax import lax
from jax.experimental import pallas as pl
from jax.experimental.pallas import tpu as pltpu
from jax.experimental.pallas import tpu_sc as plsc

_K = 3
_M = 4
_L = 16


def _build_adj_t(ei, n, e):
    info = plsc.get_sparse_core_info()
    nw = info.num_cores * info.num_subcores
    rows = n // nw
    mesh = plsc.VectorSubcoreMesh(core_axis_name="c", subcore_axis_name="s")

    @functools.partial(
        pl.kernel,
        mesh=mesh,
        out_type=jax.ShapeDtypeStruct((n, n), jnp.float32),
        scratch_types=[
            pltpu.VMEM((e,), jnp.int32),
            pltpu.VMEM((e,), jnp.int32),
            pltpu.VMEM((rows, n), jnp.float32),
            pltpu.SemaphoreType.DMA,
            pltpu.SemaphoreType.DMA,
        ],
        compiler_params=pltpu.CompilerParams(needs_layout_passes=False),
    )
    def sc_scatter(src_hbm, dst_hbm, t_hbm, src_v, dst_v, slab, sem1, sem2):
        wid = lax.axis_index("s") * info.num_cores + lax.axis_index("c")
        base = wid * rows
        cp1 = pltpu.async_copy(src_hbm, src_v, sem1)
        cp2 = pltpu.async_copy(dst_hbm, dst_v, sem2)

        zeros = jnp.zeros((_L,), jnp.float32)
        npl = n // _L

        @plsc.parallel_loop(0, rows * npl, 1, unroll=4)
        def _(j):
            slab[j // npl, pl.ds((j % npl) * _L, _L)] = zeros

        cp1.wait()
        cp2.wait()

        ones = jnp.ones((_L,), jnp.float32)
        lanes = lax.iota(jnp.int32, _L)

        for r0 in range(0, rows, _L):
            plsc.store_scatter(slab, [r0 + lanes, base + r0 + lanes], ones)

        @plsc.parallel_loop(0, e // _L, 1, unroll=8)
        def _(i):
            s = src_v[pl.ds(i * _L, _L)]
            d = dst_v[pl.ds(i * _L, _L)]
            r = d - base
            m = (r >= 0) & (r < rows)
            plsc.store_scatter(slab, [r, s], ones, mask=m)

        pltpu.sync_copy(slab, t_hbm.at[pl.ds(base, rows)])

    return sc_scatter(ei[0], ei[1])


def _tc_body(t_ref, x_ref, out_ref):
    d = x_ref.shape[1]

    xa = jnp.abs(x_ref[...])
    x2 = xa * xa
    xcat = jnp.concatenate([xa, x2, x2 * xa, x2 * x2], axis=1)
    xcat = xcat.astype(jnp.bfloat16)

    t1 = t_ref[...].astype(jnp.bfloat16)
    c2 = lax.dot_general(t1, t1, (((1,), (0,)), ((), ())),
                         preferred_element_type=jnp.float32)
    t2 = (c2 > 0.0).astype(jnp.bfloat16)
    c3 = lax.dot_general(t1, t2, (((1,), (0,)), ((), ())),
                         preferred_element_type=jnp.float32)
    t3 = (c3 > 0.0).astype(jnp.bfloat16)

    for k, t in enumerate((t1, t2, t3)):
        ok = lax.dot_general(t, xcat, (((1,), (0,)), ((), ())),
                             preferred_element_type=jnp.float32)
        for m in range(_M):
            out_ref[0, :, k, m, :] = ok[:, m * d:(m + 1) * d]


def _tc_compute(t, x2d, n, d):
    return pl.pallas_call(
        _tc_body,
        out_shape=jax.ShapeDtypeStruct((1, n, _K, _M, d), jnp.float32),
    )(t, x2d)


def kernel(x, edge_index):
    b, n, d = x.shape
    e = edge_index.shape[1]
    t = _build_adj_t(edge_index, n, e)
    outs = [_tc_compute(t, x[bi], n, d) for bi in range(b)]
    if b == 1:
        return outs[0]
    return jnp.concatenate(outs, axis=0)
